# baseline (speedup 1.0000x reference)
import functools

import jax
import jax.numpy as jnp
from jax import lax
from jax.experimental import pallas as pl
from jax.experimental.pallas import tpu as pltpu
from jax.experimental.pallas import tpu_sc as plsc

GRID2 = 196
DIM = 384
B = 32
N = 576
ROWS = B * N
NC, NS = 2, 16
NW = NC * NS
RPW = ROWS // NW
CH = 72
NCH = RPW // CH
GPR = DIM // 16

_MESH = plsc.VectorSubcoreMesh(
    core_axis_name="c", subcore_axis_name="s", num_cores=NC, num_subcores=NS
)


@functools.partial(
    pl.kernel,
    out_type=jax.ShapeDtypeStruct((ROWS, DIM), jnp.float32),
    mesh=_MESH,
    scratch_types=[
        pltpu.VMEM((1, NCH, CH), jnp.int32),
        pltpu.VMEM((2, CH, DIM), jnp.float32),
        pltpu.VMEM((2, CH, DIM), jnp.float32),
        [pltpu.SemaphoreType.DMA] * 2,
        [pltpu.SemaphoreType.DMA] * 2,
        [pltpu.SemaphoreType.DMA] * 2,
    ],
)
def _sc_kernel(in_hbm, pos_hbm, tab_hbm, out_hbm, idx_v, buf_in, buf_emb,
               sem_in, sem_emb, sem_out):
    wid = lax.axis_index("s") * NC + lax.axis_index("c")
    base = wid * RPW
    pltpu.sync_copy(pos_hbm.at[pl.ds(wid, 1)], idx_v)

    def issue(j):
        b = j % 2
        cp_i = pltpu.async_copy(
            in_hbm.at[pl.ds(base + j * CH, CH)], buf_in.at[b], sem_in[b]
        )
        cp_e = pltpu.async_copy(
            tab_hbm.at[idx_v.at[0, j]], buf_emb.at[b], sem_emb[b]
        )
        return cp_i, cp_e

    cps = issue(0)
    out_cp = [None, None]
    for j in range(NCH):
        b = j % 2
        cur = cps
        if j + 1 < NCH:
            if out_cp[1 - b] is not None:
                out_cp[1 - b].wait()
            cps = issue(j + 1)
        cur[0].wait()
        cur[1].wait()

        def row_body(r, c2, b=b):
            for g in range(GPR):
                sl = pl.ds(g * 16, 16)
                buf_in[b, r, sl] = buf_in[b, r, sl] + buf_emb[b, r, sl]
            return c2

        lax.fori_loop(0, CH, row_body, 0)
        out_cp[b] = pltpu.async_copy(
            buf_in.at[b], out_hbm.at[pl.ds(base + j * CH, CH)], sem_out[b]
        )
    out_cp[0].wait()
    out_cp[1].wait()


def kernel(inputs, inputs_positions, position_emb):
    pos = inputs_positions.astype(jnp.int32).reshape(NW, NCH, CH)
    out = _sc_kernel(
        inputs.reshape(ROWS, DIM),
        pos,
        position_emb.reshape(GRID2, DIM),
    )
    return out.reshape(B, N, DIM)


# speedup vs baseline: 1.1519x; 1.1519x over previous
import functools

import jax
import jax.numpy as jnp
from jax import lax
from jax.experimental import pallas as pl
from jax.experimental.pallas import tpu as pltpu
from jax.experimental.pallas import tpu_sc as plsc

GRID2 = 196
DIM = 384
B = 32
N = 576
ROWS = B * N
NC, NS = 2, 16
NW = NC * NS
RPW = ROWS // NW
CH = 96
NCH = RPW // CH
GPR = DIM // 16

_MESH = plsc.VectorSubcoreMesh(
    core_axis_name="c", subcore_axis_name="s", num_cores=NC, num_subcores=NS
)


@functools.partial(
    pl.kernel,
    out_type=jax.ShapeDtypeStruct((ROWS, DIM), jnp.float32),
    mesh=_MESH,
    scratch_types=[
        pltpu.VMEM((1, NCH, CH), jnp.int32),
        pltpu.VMEM((CH, DIM), jnp.float32),
        pltpu.VMEM((CH, DIM), jnp.float32),
        pltpu.SemaphoreType.DMA,
        pltpu.SemaphoreType.DMA,
    ],
)
def _sc_kernel(in_hbm, pos_hbm, tab_hbm, out_hbm, idx_v, buf_in, buf_emb,
               sem_in, sem_emb):
    wid = lax.axis_index("s") * NC + lax.axis_index("c")
    pltpu.sync_copy(pos_hbm.at[pl.ds(wid, 1)], idx_v)

    def chunk_body(j, carry):
        row0 = wid * RPW + j * CH
        cp_in = pltpu.async_copy(in_hbm.at[pl.ds(row0, CH)], buf_in, sem_in)
        cp_emb = pltpu.async_copy(tab_hbm.at[idx_v.at[0, j]], buf_emb, sem_emb)
        cp_in.wait()
        cp_emb.wait()

        def row_body(r, c2):
            for g in range(GPR):
                sl = pl.ds(g * 16, 16)
                plsc.addupdate(buf_in.at[r, sl], buf_emb[r, sl])
            return c2

        lax.fori_loop(0, CH, row_body, 0)
        pltpu.sync_copy(buf_in, out_hbm.at[pl.ds(row0, CH)])
        return carry

    lax.fori_loop(0, NCH, chunk_body, 0)


def kernel(inputs, inputs_positions, position_emb):
    pos = inputs_positions.astype(jnp.int32).reshape(NW, NCH, CH)
    out = _sc_kernel(
        inputs.reshape(ROWS, DIM),
        pos,
        position_emb.reshape(GRID2, DIM),
    )
    return out.reshape(B, N, DIM)


# speedup vs baseline: 1.1792x; 1.0236x over previous
import functools

import jax
import jax.numpy as jnp
from jax import lax
from jax.experimental import pallas as pl
from jax.experimental.pallas import tpu as pltpu
from jax.experimental.pallas import tpu_sc as plsc

GRID2 = 196
DIM = 384
B = 32
N = 576
ROWS = B * N
NC, NS = 2, 16
NW = NC * NS
RPW = ROWS // NW
CH = 96
NCH = RPW // CH
GPR = DIM // 16

_MESH = plsc.VectorSubcoreMesh(
    core_axis_name="c", subcore_axis_name="s", num_cores=NC, num_subcores=NS
)


@functools.partial(
    pl.kernel,
    out_type=jax.ShapeDtypeStruct((ROWS, DIM), jnp.float32),
    mesh=_MESH,
    scratch_types=[
        pltpu.VMEM((1, NCH, CH), jnp.int32),
        pltpu.VMEM((CH, DIM), jnp.float32),
        pltpu.VMEM((CH, DIM), jnp.float32),
        pltpu.VMEM((CH, DIM), jnp.float32),
        pltpu.SemaphoreType.DMA,
        pltpu.SemaphoreType.DMA,
        pltpu.SemaphoreType.DMA,
    ],
)
def _sc_kernel(in_hbm, pos_hbm, tab_hbm, out_hbm, idx_v, buf_in, buf_emb,
               buf_out, sem_in, sem_emb, sem_out):
    wid = lax.axis_index("s") * NC + lax.axis_index("c")
    pltpu.sync_copy(pos_hbm.at[pl.ds(wid, 1)], idx_v)
    out_cp = [None]

    for j in range(NCH):
        row0 = wid * RPW + j * CH
        cp_in = pltpu.async_copy(in_hbm.at[pl.ds(row0, CH)], buf_in, sem_in)
        cp_emb = pltpu.async_copy(tab_hbm.at[idx_v.at[0, j]], buf_emb, sem_emb)
        cp_in.wait()
        cp_emb.wait()
        if out_cp[0] is not None:
            out_cp[0].wait()

        def row_body(r, c2):
            for g in range(GPR):
                sl = pl.ds(g * 16, 16)
                buf_out[r, sl] = buf_in[r, sl] + buf_emb[r, sl]
            return c2

        lax.fori_loop(0, CH, row_body, 0)
        out_cp[0] = pltpu.async_copy(
            buf_out, out_hbm.at[pl.ds(row0, CH)], sem_out
        )
    out_cp[0].wait()


def kernel(inputs, inputs_positions, position_emb):
    pos = inputs_positions.astype(jnp.int32).reshape(NW, NCH, CH)
    out = _sc_kernel(
        inputs.reshape(ROWS, DIM),
        pos,
        position_emb.reshape(GRID2, DIM),
    )
    return out.reshape(B, N, DIM)
